# baseline (speedup 1.0000x reference)
import functools

import jax
import jax.numpy as jnp
from jax import lax
from jax.experimental import pallas as pl
from jax.experimental.pallas import tpu as pltpu
from jax.experimental.pallas import tpu_sc as plsc

B = 16384
L = 200
D = 64
G = 104
NDATA = 100
PADL = 256
DUMP = 255
NC = 2
NS = 16
NW = NC * NS
ROWS = B * L // NDATA
RPW = ROWS // NW
SUP = 8
BLK = 64


def _sc_body(ids_hbm, cmap_hbm, zeros_hbm, table_hbm, out_hbm,
             ibuf, rowsA, rowsB, cmap0, cmap1, zbuf, rtmp, rsum, acc,
             gsemA, gsemB):
    c = lax.axis_index("c")
    s = lax.axis_index("s")
    wid = s * NC + c
    base = wid * RPW

    pltpu.sync_copy(cmap_hbm.at[s, 0], cmap0)
    pltpu.sync_copy(cmap_hbm.at[s, 1], cmap1)
    pltpu.sync_copy(zeros_hbm, zbuf)
    pltpu.sync_copy(zbuf, acc.at[pl.ds(s * PADL, PADL)])

    @pl.loop(0, RPW // BLK)
    def _blk(kb):
        pltpu.sync_copy(ids_hbm.at[pl.ds(base + kb * BLK, BLK)], ibuf)

        @pl.loop(0, BLK // 2)
        def _sup(t):
            cpA = pltpu.async_copy(table_hbm.at[ibuf.at[t * 2]], rowsA, gsemA)
            cpB = pltpu.async_copy(table_hbm.at[ibuf.at[t * 2 + 1]], rowsB, gsemB)
            cpA.wait()
            pltpu.sync_copy(rowsA, acc.at[cmap0], add=True)
            cpB.wait()
            pltpu.sync_copy(rowsB, acc.at[cmap1], add=True)

    plsc.subcore_barrier()

    pltpu.sync_copy(acc.at[pl.ds(s * 16, 16)], rsum)

    @pl.loop(1, NS)
    def _red(p):
        pltpu.sync_copy(acc.at[pl.ds(p * PADL + s * 16, 16)], rtmp)
        for row in range(16):
            for k2 in range(D // 16):
                sl = (row, pl.ds(k2 * 16, 16))
                rsum[sl] = rsum[sl] + rtmp[sl]

    pltpu.sync_copy(rsum, out_hbm.at[c, pl.ds(s * 16, 16)])


_sc_embed = functools.partial(
    pl.kernel,
    out_type=jax.ShapeDtypeStruct((NC, PADL, D), jnp.float32),
    mesh=plsc.VectorSubcoreMesh(
        core_axis_name="c", subcore_axis_name="s",
        num_cores=NC, num_subcores=NS),
    compiler_params=pltpu.CompilerParams(use_tc_tiling_on_sc=False),
    scratch_types=[
        pltpu.VMEM((BLK, G), jnp.int32),
        pltpu.VMEM((G, D), jnp.float32),
        pltpu.VMEM((G, D), jnp.float32),
        pltpu.VMEM((G,), jnp.int32),
        pltpu.VMEM((G,), jnp.int32),
        pltpu.VMEM((PADL, D), jnp.float32),
        pltpu.VMEM((16, D), jnp.float32),
        pltpu.VMEM((16, D), jnp.float32),
        pltpu.VMEM_SHARED((NS * PADL, D), jnp.float32),
        pltpu.SemaphoreType.DMA,
        pltpu.SemaphoreType.DMA,
    ],
)(_sc_body)


def _tc_combine(p_ref, o_ref):
    o_ref[...] = (p_ref[0, :L, :] + p_ref[1, :L, :]) * jnp.float32(1.0 / B)


def kernel(token_ids, embedding_table):
    ids = token_ids.reshape(ROWS, NDATA)
    ids = jnp.pad(ids, ((0, 0), (0, G - NDATA)))

    j = jnp.arange(G, dtype=jnp.int32)
    phase0 = jnp.where(j < NDATA, j, DUMP)
    phase1 = jnp.where(j < NDATA, j + NDATA, DUMP)
    patt = jnp.stack([phase0, phase1])
    cmap = (jnp.arange(NS, dtype=jnp.int32) * PADL)[:, None, None] + patt[None]

    zeros = jnp.zeros((PADL, D), jnp.float32)

    partial = _sc_embed(ids, cmap, zeros, embedding_table)

    return pl.pallas_call(
        _tc_combine,
        out_shape=jax.ShapeDtypeStruct((L, D), jnp.float32),
    )(partial)
